# baseline (speedup 1.0000x reference)
import functools

import jax
import jax.numpy as jnp
from jax.experimental import pallas as pl

_P = 8
_NPACK = 12
_NREM = 4


def _body(x_ref, w8_ref, b8_ref, w4_ref, b4_ref, o_ref):
    ki = _P * 16
    ko = _P * 32
    for p in range(_NPACK):
        o_ref[:, ko * p:ko * (p + 1)] = (
            jnp.dot(x_ref[:, ki * p:ki * (p + 1)], w8_ref[p],
                    preferred_element_type=jnp.float32)
            + b8_ref[p:p + 1, :]
        )
    o_ref[:, ko * _NPACK:] = (
        jnp.dot(x_ref[:, ki * _NPACK:], w4_ref[...],
                preferred_element_type=jnp.float32)
        + b4_ref[...]
    )


def _pack_blockdiag(Wg):
    G, P, I, O = Wg.shape
    eye = jnp.eye(P, dtype=Wg.dtype)
    return (Wg[:, :, :, None, :] * eye[None, :, None, :, None]).reshape(G, P * I, P * O)


@functools.partial(jax.jit, static_argnames=("block_b",))
def _attribute_embed(x, W, b, block_b=1024):
    B, N, I = x.shape
    O = W.shape[2]
    n_full = _NPACK * _P

    W8 = _pack_blockdiag(W[:n_full].reshape(_NPACK, _P, I, O))
    W4 = _pack_blockdiag(W[n_full:].reshape(1, _NREM, I, O))[0]
    b8 = b[:n_full].reshape(_NPACK, _P * O)
    b4 = b[n_full:].reshape(1, _NREM * O)

    x2 = x.reshape(B, N * I)
    nb = B // block_b

    out2 = pl.pallas_call(
        _body,
        grid=(nb,),
        in_specs=[
            pl.BlockSpec((block_b, N * I), lambda i: (i, 0)),
            pl.BlockSpec(W8.shape, lambda i: (0, 0, 0)),
            pl.BlockSpec(b8.shape, lambda i: (0, 0)),
            pl.BlockSpec(W4.shape, lambda i: (0, 0)),
            pl.BlockSpec(b4.shape, lambda i: (0, 0)),
        ],
        out_specs=pl.BlockSpec((block_b, N * O), lambda i: (i, 0)),
        out_shape=jax.ShapeDtypeStruct((B, N * O), jnp.float32),
    )(x2, W8, b8, W4, b4)

    return out2.reshape(B, N, O)


def kernel(x, W, b):
    return _attribute_embed(x, W, b)
